# Initial kernel scaffold; baseline (speedup 1.0000x reference)
import functools

import jax
import jax.numpy as jnp
from jax import lax
from jax.experimental import pallas as pl
from jax.experimental.pallas import tpu as pltpu
from jax.experimental.pallas import tpu_sc as plsc

N = 10000
E = 320000
D = 128
G = 64
NC = 2
NS = 16
NW = NC * NS
EPT = E // NW
CHUNK = 80
NCHUNK = EPT // CHUNK
RPS = N // NS
BR = 2000
NB = N // BR

_mesh = plsc.VectorSubcoreMesh(core_axis_name="c", subcore_axis_name="s")


def _deg_body(dst_hbm, zeros_hbm, out_hbm, idx_v, ones_v, deg_sh):
    cid = lax.axis_index("c")
    sid = lax.axis_index("s")

    @pl.when(sid == 0)
    def _():
        pltpu.sync_copy(zeros_hbm, deg_sh)

    for j in range(CHUNK // 16):
        ones_v[pl.ds(j * 16, 16)] = jnp.full((16,), 1.0, jnp.float32)
    plsc.subcore_barrier()

    def step(i, _):
        base = cid * (NS * EPT) + sid * EPT + i * CHUNK
        pltpu.sync_copy(dst_hbm.at[pl.ds(base, CHUNK)], idx_v)
        pltpu.sync_copy(ones_v, deg_sh.at[idx_v], add=True)
        return 0

    lax.fori_loop(0, NCHUNK, step, 0)
    plsc.subcore_barrier()

    @pl.when(sid == 0)
    def _():
        pltpu.sync_copy(deg_sh, out_hbm.at[cid])


_deg_call = functools.partial(
    pl.kernel,
    out_type=jax.ShapeDtypeStruct((NC, N), jnp.float32),
    mesh=_mesh,
    scratch_types=[
        pltpu.VMEM((CHUNK,), jnp.int32),
        pltpu.VMEM((CHUNK,), jnp.float32),
        pltpu.VMEM_SHARED((N,), jnp.float32),
    ],
)(_deg_body)


def _agg_body(h_hbm, src_hbm, dst_hbm, zeros_hbm, out_hbm,
              sidx_v, didx_v, rows_v, acc_sh):
    cid = lax.axis_index("c")
    sid = lax.axis_index("s")

    pltpu.sync_copy(zeros_hbm.at[pl.ds(sid * RPS, RPS)],
                    acc_sh.at[pl.ds(sid * RPS, RPS)])
    plsc.subcore_barrier()

    def step(i, _):
        base = cid * (NS * EPT) + sid * EPT + i * CHUNK
        pltpu.sync_copy(src_hbm.at[pl.ds(base, CHUNK)], sidx_v)
        pltpu.sync_copy(dst_hbm.at[pl.ds(base, CHUNK)], didx_v)
        pltpu.sync_copy(h_hbm.at[sidx_v], rows_v)
        pltpu.sync_copy(rows_v, acc_sh.at[didx_v], add=True)
        return 0

    lax.fori_loop(0, NCHUNK, step, 0)
    plsc.subcore_barrier()
    pltpu.sync_copy(acc_sh.at[pl.ds(sid * RPS, RPS)],
                    out_hbm.at[cid].at[pl.ds(sid * RPS, RPS)])


_agg_call = functools.partial(
    pl.kernel,
    out_type=jax.ShapeDtypeStruct((NC, N, D), jnp.float32),
    mesh=_mesh,
    scratch_types=[
        pltpu.VMEM((CHUNK,), jnp.int32),
        pltpu.VMEM((CHUNK,), jnp.int32),
        pltpu.VMEM((CHUNK, D), jnp.float32),
        pltpu.VMEM_SHARED((N, D), jnp.float32),
    ],
)(_agg_body)


def _tc1_body(x_ref, w_ref, dga_ref, dgb_ref, hp_ref, dis_ref):
    deg = dga_ref[...] + dgb_ref[...] + 1.0
    dis = lax.rsqrt(deg)
    dis_ref[...] = dis
    h = jnp.dot(x_ref[...], w_ref[...], preferred_element_type=jnp.float32)
    hp_ref[...] = h * dis


def _tc1(x, W, degA, degB):
    return pl.pallas_call(
        _tc1_body,
        grid=(NB,),
        in_specs=[
            pl.BlockSpec((BR, D), lambda i: (i, 0)),
            pl.BlockSpec((D, D), lambda i: (0, 0)),
            pl.BlockSpec((BR, 1), lambda i: (i, 0)),
            pl.BlockSpec((BR, 1), lambda i: (i, 0)),
        ],
        out_specs=[
            pl.BlockSpec((BR, D), lambda i: (i, 0)),
            pl.BlockSpec((BR, 1), lambda i: (i, 0)),
        ],
        out_shape=[
            jax.ShapeDtypeStruct((N, D), jnp.float32),
            jax.ShapeDtypeStruct((N, 1), jnp.float32),
        ],
    )(x, W, degA, degB)


def _tc2_body(acc_ref, hp_ref, dis_ref, b_ref, w_ref, out_ref):
    acc = acc_ref[...]
    s = acc[0] + acc[1] + hp_ref[...]
    z = jnp.maximum(s * dis_ref[...] + b_ref[...], 0.0)
    h = jnp.dot(z, w_ref[...], preferred_element_type=jnp.float32)
    out_ref[...] = h * dis_ref[...]


def _tc2(acc, hp, dis, b, W):
    return pl.pallas_call(
        _tc2_body,
        grid=(NB,),
        in_specs=[
            pl.BlockSpec((NC, BR, D), lambda i: (0, i, 0)),
            pl.BlockSpec((BR, D), lambda i: (i, 0)),
            pl.BlockSpec((BR, 1), lambda i: (i, 0)),
            pl.BlockSpec((1, D), lambda i: (0, 0)),
            pl.BlockSpec((D, D), lambda i: (0, 0)),
        ],
        out_specs=pl.BlockSpec((BR, D), lambda i: (i, 0)),
        out_shape=jax.ShapeDtypeStruct((N, D), jnp.float32),
    )(acc, hp, dis, b, W)


def _tc3_body(acc_ref, hp_ref, dis_ref, b_ref, bat_ref, wfc_ref, bfc_ref,
              out_ref, sums_ref, cnts_ref):
    i = pl.program_id(0)
    acc = acc_ref[...]
    s = acc[0] + acc[1] + hp_ref[...]
    z = jnp.maximum(s * dis_ref[...] + b_ref[...], 0.0)
    gids = lax.broadcasted_iota(jnp.int32, (BR, G), 1)
    oh = (bat_ref[...] == gids).astype(jnp.float32)
    part = lax.dot_general(oh, z, (((0,), (0,)), ((), ())),
                           preferred_element_type=jnp.float32)
    cpart = lax.dot_general(oh, jnp.ones_like(z), (((0,), (0,)), ((), ())),
                            preferred_element_type=jnp.float32)

    @pl.when(i == 0)
    def _():
        sums_ref[...] = jnp.zeros_like(sums_ref)
        cnts_ref[...] = jnp.zeros_like(cnts_ref)

    sums_ref[...] += part
    cnts_ref[...] += cpart

    @pl.when(i == NB - 1)
    def _():
        emb = sums_ref[...] / jnp.maximum(cnts_ref[...], 1.0)
        out_ref[...] = jnp.dot(emb, wfc_ref[...],
                               preferred_element_type=jnp.float32) + bfc_ref[...]


def _tc3(acc, hp, dis, b, batch2d, Wfc, bfc):
    return pl.pallas_call(
        _tc3_body,
        grid=(NB,),
        in_specs=[
            pl.BlockSpec((NC, BR, D), lambda i: (0, i, 0)),
            pl.BlockSpec((BR, D), lambda i: (i, 0)),
            pl.BlockSpec((BR, 1), lambda i: (i, 0)),
            pl.BlockSpec((1, D), lambda i: (0, 0)),
            pl.BlockSpec((BR, 1), lambda i: (i, 0)),
            pl.BlockSpec((D, D), lambda i: (0, 0)),
            pl.BlockSpec((1, D), lambda i: (0, 0)),
        ],
        out_specs=pl.BlockSpec((G, D), lambda i: (0, 0)),
        out_shape=jax.ShapeDtypeStruct((G, D), jnp.float32),
        scratch_shapes=[
            pltpu.VMEM((G, D), jnp.float32),
            pltpu.VMEM((G, D), jnp.float32),
        ],
    )(acc, hp, dis, b, batch2d, Wfc, bfc)


@jax.jit
def kernel(x, edge_index, batch, W1, b1, W2, b2, Wfc, bfc):
    src = edge_index[0]
    dst = edge_index[1]
    zeros1 = jnp.zeros((N,), jnp.float32)
    zeros2 = jnp.zeros((N, D), jnp.float32)

    deg = _deg_call(dst, zeros1)
    degA = deg[0].reshape(N, 1)
    degB = deg[1].reshape(N, 1)

    h1p, dis = _tc1(x, W1, degA, degB)
    acc1 = _agg_call(h1p, src, dst, zeros2)
    h2p = _tc2(acc1, h1p, dis, b1.reshape(1, D), W2)
    acc2 = _agg_call(h2p, src, dst, zeros2)
    return _tc3(acc2, h2p, dis, b2.reshape(1, D), batch.reshape(N, 1),
                Wfc, bfc.reshape(1, D))


# speedup vs baseline: 13.1966x; 13.1966x over previous
import functools

import jax
import jax.numpy as jnp
from jax import lax
from jax.experimental import pallas as pl
from jax.experimental.pallas import tpu as pltpu
from jax.experimental.pallas import tpu_sc as plsc

N = 10000
E = 320000
D = 128
G = 64
NC = 2
NS = 16
NW = NC * NS
EPT = E // NW
CHUNK = 80
NCHUNK = EPT // CHUNK
NP = 10240
RPS = 624
RPS_LAST = N - RPS * (NS - 1)
BR = 2000
NB = N // BR

_mesh = plsc.VectorSubcoreMesh(core_axis_name="c", subcore_axis_name="s")


def _copy_rows(src, dst, sid, add=False):
    @pl.when(sid < NS - 1)
    def _():
        pltpu.sync_copy(src.at[pl.ds(sid * RPS, RPS)],
                        dst.at[pl.ds(sid * RPS, RPS)], add=add)

    @pl.when(sid == NS - 1)
    def _():
        pltpu.sync_copy(src.at[pl.ds((NS - 1) * RPS, RPS_LAST)],
                        dst.at[pl.ds((NS - 1) * RPS, RPS_LAST)], add=add)


def _deg_body(dst_hbm, zeros_hbm, out_hbm, idx_v, ones_v, deg_sh):
    cid = lax.axis_index("c")
    sid = lax.axis_index("s")

    @pl.when(sid == 0)
    def _():
        pltpu.sync_copy(zeros_hbm, deg_sh)

    for j in range(CHUNK // 16):
        ones_v[pl.ds(j * 16, 16)] = jnp.full((16,), 1.0, jnp.float32)
    plsc.subcore_barrier()

    def step(i, _):
        base = cid * (NS * EPT) + sid * EPT + i * CHUNK
        pltpu.sync_copy(dst_hbm.at[pl.ds(base, CHUNK)], idx_v)
        pltpu.sync_copy(ones_v, deg_sh.at[idx_v], add=True)
        return 0

    lax.fori_loop(0, NCHUNK, step, 0)
    plsc.subcore_barrier()

    @pl.when(sid == 0)
    def _():
        pltpu.sync_copy(deg_sh, out_hbm.at[pl.ds(cid * NP, NP)])


_deg_call = functools.partial(
    pl.kernel,
    out_type=jax.ShapeDtypeStruct((NC * NP,), jnp.float32),
    mesh=_mesh,
    scratch_types=[
        pltpu.VMEM((CHUNK,), jnp.int32),
        pltpu.VMEM((CHUNK,), jnp.float32),
        pltpu.VMEM_SHARED((NP,), jnp.float32),
    ],
)(_deg_body)


def _agg_body(h_hbm, src_hbm, dst_hbm, zeros_hbm, out_hbm,
              sidx_v, didx_v, rows_v, acc_sh):
    cid = lax.axis_index("c")
    sid = lax.axis_index("s")

    _copy_rows(zeros_hbm, acc_sh, sid)
    plsc.subcore_barrier()

    def step(i, _):
        base = cid * (NS * EPT) + sid * EPT + i * CHUNK
        pltpu.sync_copy(src_hbm.at[pl.ds(base, CHUNK)], sidx_v)
        pltpu.sync_copy(dst_hbm.at[pl.ds(base, CHUNK)], didx_v)
        pltpu.sync_copy(h_hbm.at[sidx_v], rows_v)
        pltpu.sync_copy(rows_v, acc_sh.at[didx_v], add=True)
        return 0

    lax.fori_loop(0, NCHUNK, step, 0)
    plsc.subcore_barrier()
    _copy_rows(acc_sh, out_hbm.at[cid], sid)


_agg_call = functools.partial(
    pl.kernel,
    out_type=jax.ShapeDtypeStruct((NC, N, D), jnp.float32),
    mesh=_mesh,
    scratch_types=[
        pltpu.VMEM((CHUNK,), jnp.int32),
        pltpu.VMEM((CHUNK,), jnp.int32),
        pltpu.VMEM((CHUNK, D), jnp.float32),
        pltpu.VMEM_SHARED((N, D), jnp.float32),
    ],
)(_agg_body)


def _tc1_body(x_ref, w_ref, dga_ref, dgb_ref, hp_ref, dis_ref):
    deg = dga_ref[...] + dgb_ref[...] + 1.0
    dis = lax.rsqrt(deg)
    dis_ref[...] = dis
    h = jnp.dot(x_ref[...], w_ref[...], preferred_element_type=jnp.float32)
    hp_ref[...] = h * dis


def _tc1(x, W, degA, degB):
    return pl.pallas_call(
        _tc1_body,
        grid=(NB,),
        in_specs=[
            pl.BlockSpec((BR, D), lambda i: (i, 0)),
            pl.BlockSpec((D, D), lambda i: (0, 0)),
            pl.BlockSpec((BR, 1), lambda i: (i, 0)),
            pl.BlockSpec((BR, 1), lambda i: (i, 0)),
        ],
        out_specs=[
            pl.BlockSpec((BR, D), lambda i: (i, 0)),
            pl.BlockSpec((BR, 1), lambda i: (i, 0)),
        ],
        out_shape=[
            jax.ShapeDtypeStruct((N, D), jnp.float32),
            jax.ShapeDtypeStruct((N, 1), jnp.float32),
        ],
    )(x, W, degA, degB)


def _tc2_body(acc_ref, hp_ref, dis_ref, b_ref, w_ref, out_ref):
    acc = acc_ref[...]
    s = acc[0] + acc[1] + hp_ref[...]
    z = jnp.maximum(s * dis_ref[...] + b_ref[...], 0.0)
    h = jnp.dot(z, w_ref[...], preferred_element_type=jnp.float32)
    out_ref[...] = h * dis_ref[...]


def _tc2(acc, hp, dis, b, W):
    return pl.pallas_call(
        _tc2_body,
        grid=(NB,),
        in_specs=[
            pl.BlockSpec((NC, BR, D), lambda i: (0, i, 0)),
            pl.BlockSpec((BR, D), lambda i: (i, 0)),
            pl.BlockSpec((BR, 1), lambda i: (i, 0)),
            pl.BlockSpec((1, D), lambda i: (0, 0)),
            pl.BlockSpec((D, D), lambda i: (0, 0)),
        ],
        out_specs=pl.BlockSpec((BR, D), lambda i: (i, 0)),
        out_shape=jax.ShapeDtypeStruct((N, D), jnp.float32),
    )(acc, hp, dis, b, W)


def _tc3_body(acc_ref, hp_ref, dis_ref, b_ref, bat_ref, wfc_ref, bfc_ref,
              out_ref, sums_ref, cnts_ref):
    i = pl.program_id(0)
    acc = acc_ref[...]
    s = acc[0] + acc[1] + hp_ref[...]
    z = jnp.maximum(s * dis_ref[...] + b_ref[...], 0.0)
    gids = lax.broadcasted_iota(jnp.int32, (BR, G), 1)
    oh = (bat_ref[...] == gids).astype(jnp.float32)
    part = lax.dot_general(oh, z, (((0,), (0,)), ((), ())),
                           preferred_element_type=jnp.float32)
    cpart = lax.dot_general(oh, jnp.ones_like(z), (((0,), (0,)), ((), ())),
                            preferred_element_type=jnp.float32)

    @pl.when(i == 0)
    def _():
        sums_ref[...] = jnp.zeros_like(sums_ref)
        cnts_ref[...] = jnp.zeros_like(cnts_ref)

    sums_ref[...] += part
    cnts_ref[...] += cpart

    @pl.when(i == NB - 1)
    def _():
        emb = sums_ref[...] / jnp.maximum(cnts_ref[...], 1.0)
        out_ref[...] = jnp.dot(emb, wfc_ref[...],
                               preferred_element_type=jnp.float32) + bfc_ref[...]


def _tc3(acc, hp, dis, b, batch2d, Wfc, bfc):
    return pl.pallas_call(
        _tc3_body,
        grid=(NB,),
        in_specs=[
            pl.BlockSpec((NC, BR, D), lambda i: (0, i, 0)),
            pl.BlockSpec((BR, D), lambda i: (i, 0)),
            pl.BlockSpec((BR, 1), lambda i: (i, 0)),
            pl.BlockSpec((1, D), lambda i: (0, 0)),
            pl.BlockSpec((BR, 1), lambda i: (i, 0)),
            pl.BlockSpec((D, D), lambda i: (0, 0)),
            pl.BlockSpec((1, D), lambda i: (0, 0)),
        ],
        out_specs=pl.BlockSpec((G, D), lambda i: (0, 0)),
        out_shape=jax.ShapeDtypeStruct((G, D), jnp.float32),
        scratch_shapes=[
            pltpu.VMEM((G, D), jnp.float32),
            pltpu.VMEM((G, D), jnp.float32),
        ],
    )(acc, hp, dis, b, batch2d, Wfc, bfc)


@jax.jit
def kernel(x, edge_index, batch, W1, b1, W2, b2, Wfc, bfc):
    src = edge_index[0]
    dst = edge_index[1]
    zeros1 = jnp.zeros((NP,), jnp.float32)
    zeros2 = jnp.zeros((N, D), jnp.float32)

    deg = _deg_call(dst, zeros1)
    degA = deg[:N].reshape(N, 1)
    degB = deg[NP:NP + N].reshape(N, 1)

    h1p, dis = _tc1(x, W1, degA, degB)
    acc1 = _agg_call(h1p, src, dst, zeros2)
    h2p = _tc2(acc1, h1p, dis, b1.reshape(1, D), W2)
    acc2 = _agg_call(h2p, src, dst, zeros2)
    return _tc3(acc2, h2p, dis, b2.reshape(1, D), batch.reshape(N, 1),
                Wfc, bfc.reshape(1, D))
